# Initial kernel scaffold; baseline (speedup 1.0000x reference)
import jax
import jax.numpy as jnp
from jax.experimental import pallas as pl


def kernel(x, edge_index, W1_rel, b1, W1_root, W2_rel, b2, W2_root, Wfc1, bfc1, Wfc2, bfc2):
    raise NotImplementedError("write your pallas kernel here")


# speedup vs baseline: 4.2868x; 4.2868x over previous
import functools

import jax
import jax.numpy as jnp
from jax import lax
from jax.experimental import pallas as pl
from jax.experimental.pallas import tpu as pltpu
from jax.experimental.pallas import tpu_sc as plsc

N_NODES = 10000
N_EDGES = 320000
D = 128
N_ASSETS = 512

NC = 2
NS = 16
CHUNK = 128
CHUNKS_PER_WORKER = -(-N_EDGES // (NC * NS * CHUNK))
EDGES_PER_WORKER = CHUNKS_PER_WORKER * CHUNK
N_EDGES_PAD = NC * NS * EDGES_PER_WORKER
ACC_ROWS = 10240
ROWS_PER_SUB_ZERO = ACC_ROWS // NS
ROWS_PER_SUB_OUT = N_NODES // NS

BLK = 1000


def _sc_scatter_body(y_hbm, src_hbm, dst_hbm, out_hbm,
                     src_v, dst_v, rows_v, acc_sh, sem):
    c = lax.axis_index("c")
    s = lax.axis_index("s")

    def _zrow(r, carry):
        for k in range(D // 16):
            rows_v[r, pl.ds(k * 16, 16)] = jnp.zeros((16,), jnp.float32)
        return carry
    lax.fori_loop(0, CHUNK, _zrow, 0)
    for t in range(ROWS_PER_SUB_ZERO // CHUNK):
        pltpu.sync_copy(rows_v, acc_sh.at[pl.ds(s * ROWS_PER_SUB_ZERO + t * CHUNK, CHUNK)])
    plsc.subcore_barrier()

    pltpu.sync_copy(src_hbm.at[c, s], src_v)
    pltpu.sync_copy(dst_hbm.at[c, s], dst_v)

    def _edge(j, carry):
        pltpu.async_copy(y_hbm.at[src_v.at[j]], rows_v, sem).wait()
        pltpu.sync_copy(rows_v, acc_sh.at[dst_v.at[j]], add=True)
        return carry
    lax.fori_loop(0, CHUNKS_PER_WORKER, _edge, 0)
    plsc.subcore_barrier()

    for t in range(ROWS_PER_SUB_ZERO // CHUNK):
        r0 = s * ROWS_PER_SUB_ZERO + t * CHUNK
        pltpu.sync_copy(acc_sh.at[pl.ds(r0, CHUNK)], rows_v)
        pltpu.sync_copy(rows_v, out_hbm.at[c, pl.ds(r0, CHUNK)])


@functools.cache
def _sc_scatter_kernel():
    return pl.kernel(
        _sc_scatter_body,
        out_type=jax.ShapeDtypeStruct((NC, ACC_ROWS, D), jnp.float32),
        mesh=plsc.VectorSubcoreMesh(core_axis_name="c", subcore_axis_name="s"),
        scratch_types=[
            pltpu.VMEM((CHUNKS_PER_WORKER, CHUNK), jnp.int32),
            pltpu.VMEM((CHUNKS_PER_WORKER, CHUNK), jnp.int32),
            pltpu.VMEM((CHUNK, D), jnp.float32),
            pltpu.VMEM_SHARED((ACC_ROWS, D), jnp.float32),
            pltpu.SemaphoreType.DMA,
        ],
    )


def _sc_scatter(y, srcp, dstp):
    return _sc_scatter_kernel()(y, srcp, dstp)


def _tc_pre_body(x_ref, wrel_ref, wroot_ref, y_ref, xr_ref):
    xb = x_ref[...]
    y_ref[...] = jnp.dot(xb, wrel_ref[...], preferred_element_type=jnp.float32)
    xr_ref[...] = jnp.dot(xb, wroot_ref[...], preferred_element_type=jnp.float32)


def _tc_pre(x, w_rel, w_root):
    return pl.pallas_call(
        _tc_pre_body,
        grid=(N_NODES // BLK,),
        in_specs=[
            pl.BlockSpec((BLK, D), lambda i: (i, 0)),
            pl.BlockSpec((D, D), lambda i: (0, 0)),
            pl.BlockSpec((D, D), lambda i: (0, 0)),
        ],
        out_specs=[
            pl.BlockSpec((BLK, D), lambda i: (i, 0)),
            pl.BlockSpec((BLK, D), lambda i: (i, 0)),
        ],
        out_shape=[
            jax.ShapeDtypeStruct((N_NODES, D), jnp.float32),
            jax.ShapeDtypeStruct((N_NODES, D), jnp.float32),
        ],
    )(x, w_rel, w_root)


def _tc_mid_body(agg_ref, xr_ref, b_ref, wrel_ref, wroot_ref, y_ref, hr_ref):
    h = jnp.maximum(agg_ref[0] + agg_ref[1] + xr_ref[...] + b_ref[...], 0.0)
    y_ref[...] = jnp.dot(h, wrel_ref[...], preferred_element_type=jnp.float32)
    hr_ref[...] = jnp.dot(h, wroot_ref[...], preferred_element_type=jnp.float32)


def _tc_mid(agg, xr, b, w_rel, w_root):
    return pl.pallas_call(
        _tc_mid_body,
        grid=(N_NODES // BLK,),
        in_specs=[
            pl.BlockSpec((NC, BLK, D), lambda i: (0, i, 0)),
            pl.BlockSpec((BLK, D), lambda i: (i, 0)),
            pl.BlockSpec((1, D), lambda i: (0, 0)),
            pl.BlockSpec((D, D), lambda i: (0, 0)),
            pl.BlockSpec((D, D), lambda i: (0, 0)),
        ],
        out_specs=[
            pl.BlockSpec((BLK, D), lambda i: (i, 0)),
            pl.BlockSpec((BLK, D), lambda i: (i, 0)),
        ],
        out_shape=[
            jax.ShapeDtypeStruct((N_NODES, D), jnp.float32),
            jax.ShapeDtypeStruct((N_NODES, D), jnp.float32),
        ],
    )(agg, xr, b, w_rel, w_root)


def _tc_colsum_body(agg_ref, hr_ref, b_ref, out_ref):
    i = pl.program_id(0)

    @pl.when(i == 0)
    def _():
        out_ref[...] = jnp.zeros_like(out_ref)

    h = jnp.maximum(agg_ref[0] + agg_ref[1] + hr_ref[...] + b_ref[...], 0.0)
    out_ref[...] += jnp.sum(h, axis=0, keepdims=True)


def _tc_colsum(agg, hr, b):
    return pl.pallas_call(
        _tc_colsum_body,
        grid=(N_NODES // BLK,),
        in_specs=[
            pl.BlockSpec((NC, BLK, D), lambda i: (0, i, 0)),
            pl.BlockSpec((BLK, D), lambda i: (i, 0)),
            pl.BlockSpec((1, D), lambda i: (0, 0)),
        ],
        out_specs=pl.BlockSpec((1, D), lambda i: (0, 0)),
        out_shape=jax.ShapeDtypeStruct((1, D), jnp.float32),
    )(agg, hr, b)


def _tc_head_body(cs_ref, w1_ref, b1_ref, w2_ref, b2_ref, out_ref):
    pooled = cs_ref[...] * (1.0 / N_NODES)
    o = jnp.maximum(jnp.dot(pooled, w1_ref[...], preferred_element_type=jnp.float32)
                    + b1_ref[...], 0.0)
    logits = jnp.dot(o, w2_ref[...], preferred_element_type=jnp.float32) + b2_ref[...]
    m = jnp.max(logits, axis=-1, keepdims=True)
    e = jnp.exp(logits - m)
    out_ref[...] = e / jnp.sum(e, axis=-1, keepdims=True)


def _tc_head(colsum, w1, b1, w2, b2):
    return pl.pallas_call(
        _tc_head_body,
        out_shape=jax.ShapeDtypeStruct((1, N_ASSETS), jnp.float32),
    )(colsum, w1, b1, w2, b2)


def kernel(x, edge_index, W1_rel, b1, W1_root, W2_rel, b2, W2_root,
           Wfc1, bfc1, Wfc2, bfc2):
    src = edge_index[0].astype(jnp.int32)
    dst = edge_index[1].astype(jnp.int32)
    npad = N_EDGES_PAD - N_EDGES
    srcp = jnp.concatenate([src, jnp.zeros((npad,), jnp.int32)]).reshape(
        NC, NS, CHUNKS_PER_WORKER, CHUNK)
    dstp = jnp.concatenate([dst, jnp.full((npad,), N_NODES, jnp.int32)]).reshape(
        NC, NS, CHUNKS_PER_WORKER, CHUNK)

    y1, xr1 = _tc_pre(x, W1_rel, W1_root)
    agg1 = _sc_scatter(y1, srcp, dstp)
    y2, hr2 = _tc_mid(agg1, xr1, b1.reshape(1, D), W2_rel, W2_root)
    agg2 = _sc_scatter(y2, srcp, dstp)
    colsum = _tc_colsum(agg2, hr2, b2.reshape(1, D))
    return _tc_head(colsum, Wfc1, bfc1.reshape(1, D), Wfc2, bfc2.reshape(1, N_ASSETS))
